# baseline (speedup 1.0000x reference)
import functools

import jax
import jax.numpy as jnp
from jax import lax
from jax.experimental import pallas as pl
from jax.experimental.pallas import tpu as pltpu
from jax.experimental.pallas import tpu_sc as plsc

NW = 32
_PARAMS = pltpu.CompilerParams(
    use_tc_tiling_on_sc=False, needs_layout_passes=False)
_MESH = dict(core_axis_name="c", subcore_axis_name="s")


def _wid():
    return lax.axis_index("s") * 2 + lax.axis_index("c")


def _i16(v):
    return jnp.full((16,), v, jnp.int32)


def _expand_verts(verts_flat, Vp):
    mv = Vp // NW
    mesh = plsc.VectorSubcoreMesh(**_MESH)

    @functools.partial(
        pl.kernel,
        mesh=mesh,
        out_type=jax.ShapeDtypeStruct((Vp, 8), jnp.float32),
        compiler_params=_PARAMS,
        scratch_types=[
            pltpu.VMEM((3 * mv,), jnp.float32),
            pltpu.VMEM((mv, 8), jnp.float32),
        ],
    )
    def expand(vf_hbm, v8_hbm, vin_v, v8_v):
        base = _wid() * mv

        def group(g, _):
            lanes = g * 16 + lax.iota(jnp.int32, 16)
            for c in range(3):
                val = plsc.load_gather(vin_v, [3 * lanes + _i16(c)])
                plsc.store_scatter(v8_v, [lanes, _i16(c)], val)
            return 0

        pltpu.sync_copy(vf_hbm.at[pl.ds(3 * base, 3 * mv)], vin_v)
        lax.fori_loop(0, mv // 16, group, 0)
        pltpu.sync_copy(v8_v, v8_hbm.at[pl.ds(base, mv)])

    return expand(verts_flat)


def _build_fv(faces_flat, verts8, Fp):
    mf = Fp // NW
    rows3 = 3 * mf
    hf = mf // 2
    mesh = plsc.VectorSubcoreMesh(**_MESH)

    @functools.partial(
        pl.kernel,
        mesh=mesh,
        out_type=jax.ShapeDtypeStruct((Fp, 16), jnp.float32),
        compiler_params=_PARAMS,
        scratch_types=[
            pltpu.VMEM((rows3,), jnp.int32),
            pltpu.VMEM((rows3, 8), jnp.float32),
            pltpu.VMEM((hf, 16), jnp.float32),
            pltpu.SemaphoreType.DMA,
        ],
    )
    def build(ff_hbm, v8_hbm, fv_hbm, idx_v, rows_v, fv_v, sem):
        base = _wid() * mf
        pltpu.sync_copy(ff_hbm.at[pl.ds(3 * base, rows3)], idx_v)
        pltpu.async_copy(v8_hbm.at[idx_v], rows_v, sem).wait()
        for h in range(2):
            def group(g, _):
                lanes = g * 16 + lax.iota(jnp.int32, 16)
                rbase = 3 * (h * hf + lanes)
                for j in range(3):
                    for c in range(3):
                        val = plsc.load_gather(rows_v, [rbase + _i16(j), _i16(c)])
                        plsc.store_scatter(fv_v, [lanes, _i16(3 * j + c)], val)
                    ids = plsc.load_gather(idx_v, [rbase + _i16(j)])
                    plsc.store_scatter(fv_v, [lanes, _i16(9 + j)],
                                       plsc.bitcast(ids, jnp.float32))
                return 0

            lax.fori_loop(0, hf // 16, group, 0)
            pltpu.sync_copy(fv_v, fv_hbm.at[pl.ds(base + h * hf, hf)])

    return build(faces_flat, verts8)


def _shade(pix, bary_flat, fv, N, m):
    n_per = N // NW
    mesh = plsc.VectorSubcoreMesh(**_MESH)

    @functools.partial(
        pl.kernel,
        mesh=mesh,
        out_type=(
            jax.ShapeDtypeStruct((4 * N,), jnp.float32),
            jax.ShapeDtypeStruct((3 * N,), jnp.int32),
        ),
        compiler_params=_PARAMS,
        scratch_types=[
            pltpu.VMEM((m,), jnp.int32),
            pltpu.VMEM((m, 16), jnp.float32),
            pltpu.VMEM((3 * m,), jnp.float32),
            pltpu.VMEM((4 * m,), jnp.float32),
            pltpu.VMEM((3 * m,), jnp.int32),
            pltpu.SemaphoreType.DMA,
        ],
    )
    def shade(pix_hbm, bary_hbm, fv_hbm, res_hbm, vfo_hbm,
              pix_v, fv_v, bary_v, res_v, vf_v, sem):
        wbase = _wid() * n_per

        def chunk(i, _):
            base = wbase + i * m
            pltpu.sync_copy(pix_hbm.at[pl.ds(base, m)], pix_v)
            cp_fv = pltpu.async_copy(fv_hbm.at[pix_v], fv_v, sem)
            pltpu.sync_copy(bary_hbm.at[pl.ds(3 * base, 3 * m)], bary_v)
            cp_fv.wait()

            def group(g, _):
                s = g * 16
                rows = s + lax.iota(jnp.int32, 16)
                rows3 = 3 * rows
                rows4 = 4 * rows
                pv = pix_v[pl.ds(s, 16)]
                b = [plsc.load_gather(bary_v, [rows3 + _i16(j)]) for j in range(3)]
                for c in range(3):
                    acc = b[0] * plsc.load_gather(fv_v, [rows, _i16(c)])
                    for j in (1, 2):
                        acc = acc + b[j] * plsc.load_gather(
                            fv_v, [rows, _i16(3 * j + c)])
                    plsc.store_scatter(res_v, [rows4 + _i16(c)], acc)
                alpha = jnp.where(pv != -1, 1.0, 0.0).astype(jnp.float32)
                plsc.store_scatter(res_v, [rows4 + _i16(3)], alpha)
                for j in range(3):
                    ids = plsc.bitcast(
                        plsc.load_gather(fv_v, [rows, _i16(9 + j)]), jnp.int32)
                    plsc.store_scatter(vf_v, [rows3 + _i16(j)], ids)
                return 0

            lax.fori_loop(0, m // 16, group, 0)
            pltpu.sync_copy(res_v, res_hbm.at[pl.ds(4 * base, 4 * m)])
            pltpu.sync_copy(vf_v, vfo_hbm.at[pl.ds(3 * base, 3 * m)])
            return 0

        lax.fori_loop(0, n_per // m, chunk, 0)

    return shade(pix, bary_flat, fv)


def kernel(pix_to_face, bary_coords, faces, verts):
    B, H, W, _ = pix_to_face.shape
    N = B * H * W
    Fn = faces.shape[0]
    Vn = verts.shape[0]

    pix = pix_to_face.reshape(N)
    bary_flat = bary_coords.reshape(3 * N)

    Fp = -(-Fn // (NW * 32)) * (NW * 32)
    Vp = -(-Vn // (NW * 32)) * (NW * 32)
    faces_flat = jnp.pad(faces, ((0, Fp - Fn), (0, 0))).reshape(3 * Fp)
    verts_flat = jnp.pad(verts, ((0, Vp - Vn), (0, 0))).reshape(3 * Vp)

    verts8 = _expand_verts(verts_flat, Vp)
    fv = _build_fv(faces_flat, verts8, Fp)
    res, vf = _shade(pix, bary_flat, fv, N, 2048)

    results = res.reshape(B, H, W, 4)
    vertex_faces = vf.reshape(B, H, W, 3)
    return (results, vertex_faces, bary_coords.reshape(B, H, W, 3))


# speedup vs baseline: 12.1214x; 12.1214x over previous
import functools

import jax
import jax.numpy as jnp
from jax import lax
from jax.experimental import pallas as pl
from jax.experimental.pallas import tpu as pltpu
from jax.experimental.pallas import tpu_sc as plsc

NW = 32
_PARAMS = pltpu.CompilerParams(
    use_tc_tiling_on_sc=False, needs_layout_passes=False)
_MESH = dict(core_axis_name="c", subcore_axis_name="s")


def _wid():
    return lax.axis_index("s") * 2 + lax.axis_index("c")


def _i16(v):
    return jnp.full((16,), v, jnp.int32)


def _expand_verts(verts_flat, Vp):
    mv = Vp // NW
    mesh = plsc.VectorSubcoreMesh(**_MESH)

    @functools.partial(
        pl.kernel,
        mesh=mesh,
        out_type=jax.ShapeDtypeStruct((Vp, 8), jnp.float32),
        compiler_params=_PARAMS,
        scratch_types=[
            pltpu.VMEM((3 * mv,), jnp.float32),
            pltpu.VMEM((mv, 8), jnp.float32),
        ],
    )
    def expand(vf_hbm, v8_hbm, vin_v, v8_v):
        base = _wid() * mv

        def group(g, _):
            lanes = g * 16 + lax.iota(jnp.int32, 16)
            for c in range(3):
                val = plsc.load_gather(vin_v, [3 * lanes + _i16(c)])
                plsc.store_scatter(v8_v, [lanes, _i16(c)], val)
            return 0

        pltpu.sync_copy(vf_hbm.at[pl.ds(3 * base, 3 * mv)], vin_v)
        lax.fori_loop(0, mv // 16, group, 0)
        pltpu.sync_copy(v8_v, v8_hbm.at[pl.ds(base, mv)])

    return expand(verts_flat)


def _build_fv(faces_flat, verts8, Fp):
    mf = Fp // NW
    rows3 = 3 * mf
    hf = mf // 2
    mesh = plsc.VectorSubcoreMesh(**_MESH)

    @functools.partial(
        pl.kernel,
        mesh=mesh,
        out_type=jax.ShapeDtypeStruct((Fp, 16), jnp.float32),
        compiler_params=_PARAMS,
        scratch_types=[
            pltpu.VMEM((rows3,), jnp.int32),
            pltpu.VMEM((rows3, 8), jnp.float32),
            pltpu.VMEM((hf, 16), jnp.float32),
            pltpu.SemaphoreType.DMA,
        ],
    )
    def build(ff_hbm, v8_hbm, fv_hbm, idx_v, rows_v, fv_v, sem):
        base = _wid() * mf
        pltpu.sync_copy(ff_hbm.at[pl.ds(3 * base, rows3)], idx_v)
        pltpu.async_copy(v8_hbm.at[idx_v], rows_v, sem).wait()
        for h in range(2):
            def group(g, _):
                lanes = g * 16 + lax.iota(jnp.int32, 16)
                rbase = 3 * (h * hf + lanes)
                for j in range(3):
                    for c in range(3):
                        val = plsc.load_gather(rows_v, [rbase + _i16(j), _i16(c)])
                        plsc.store_scatter(fv_v, [lanes, _i16(3 * j + c)], val)
                    ids = plsc.load_gather(idx_v, [rbase + _i16(j)])
                    plsc.store_scatter(fv_v, [lanes, _i16(9 + j)],
                                       plsc.bitcast(ids, jnp.float32))
                return 0

            lax.fori_loop(0, hf // 16, group, 0)
            pltpu.sync_copy(fv_v, fv_hbm.at[pl.ds(base + h * hf, hf)])

    return build(faces_flat, verts8)


def _shade(pix, bary_lin, fv, N, HW):
    m = 4096
    n_per = N // NW
    mesh = plsc.VectorSubcoreMesh(**_MESH)

    @functools.partial(
        pl.kernel,
        mesh=mesh,
        out_type=(
            jax.ShapeDtypeStruct((4 * N,), jnp.float32),
            jax.ShapeDtypeStruct((3 * N,), jnp.int32),
            jax.ShapeDtypeStruct((3 * N,), jnp.float32),
        ),
        compiler_params=_PARAMS,
        scratch_types=[
            pltpu.VMEM((m,), jnp.int32),
            pltpu.VMEM((m, 16), jnp.float32),
            pltpu.VMEM((3 * m,), jnp.float32),
            pltpu.VMEM((4 * m,), jnp.float32),
            pltpu.VMEM((3 * m,), jnp.int32),
            pltpu.VMEM((3 * m,), jnp.float32),
            pltpu.SemaphoreType.DMA,
        ],
    )
    def shade(pix_hbm, bary_hbm, fv_hbm, res_hbm, vfo_hbm, bq_hbm,
              pix_v, fv_v, bary_v, res_v, vf_v, bq_v, sem):
        wbase = _wid() * n_per

        def chunk(i, _):
            base = wbase + i * m
            pltpu.sync_copy(pix_hbm.at[pl.ds(base, m)], pix_v)
            cp_fv = pltpu.async_copy(fv_hbm.at[pix_v], fv_v, sem)
            pltpu.sync_copy(bary_hbm.at[pl.ds(3 * base, 3 * m)], bary_v)
            cp_fv.wait()

            def group(g, _):
                s = g * 16
                hh = s // 512
                sw = s % 512
                wt = sw // 128
                w7 = sw % 128
                rows = s + lax.iota(jnp.int32, 16)
                pv = pix_v[pl.ds(s, 16)]
                pl_off = wt * 1024 + hh * 128 + w7
                b = []
                for j in range(3):
                    bj = bary_v[pl.ds(hh * 1536 + j * 512 + sw, 16)]
                    b.append(bj)
                    bq_v[pl.ds(j * 4096 + pl_off, 16)] = bj
                for c in range(3):
                    acc = b[0] * plsc.load_gather(fv_v, [rows, _i16(c)])
                    for j in (1, 2):
                        acc = acc + b[j] * plsc.load_gather(
                            fv_v, [rows, _i16(3 * j + c)])
                    res_v[pl.ds(hh * 2048 + wt * 512 + c * 128 + w7, 16)] = acc
                alpha = jnp.where(pv != -1, 1.0, 0.0).astype(jnp.float32)
                res_v[pl.ds(hh * 2048 + wt * 512 + 3 * 128 + w7, 16)] = alpha
                for j in range(3):
                    ids = plsc.bitcast(
                        plsc.load_gather(fv_v, [rows, _i16(9 + j)]), jnp.int32)
                    vf_v[pl.ds(j * 4096 + pl_off, 16)] = ids
                return 0

            lax.fori_loop(0, m // 16, group, 0)
            pltpu.sync_copy(res_v, res_hbm.at[pl.ds(4 * base, 4 * m)])
            b_idx = base // HW
            inb = base % HW
            for j in range(3):
                dst = (3 * b_idx + j) * HW + inb
                pltpu.sync_copy(vf_v.at[pl.ds(j * m, m)],
                                vfo_hbm.at[pl.ds(dst, m)])
                pltpu.sync_copy(bq_v.at[pl.ds(j * m, m)],
                                bq_hbm.at[pl.ds(dst, m)])
            return 0

        lax.fori_loop(0, n_per // m, chunk, 0)

    return shade(pix, bary_lin, fv)


def kernel(pix_to_face, bary_coords, faces, verts):
    B, H, W, _ = pix_to_face.shape
    N = B * H * W
    HW = H * W
    Fn = faces.shape[0]
    Vn = verts.shape[0]

    pix = pix_to_face.reshape(N)
    bary_lin = bary_coords.transpose(0, 1, 4, 3, 2).reshape(3 * N)

    Fp = -(-Fn // (NW * 32)) * (NW * 32)
    Vp = -(-Vn // (NW * 32)) * (NW * 32)
    faces_flat = jnp.pad(faces, ((0, Fp - Fn), (0, 0))).reshape(3 * Fp)
    verts_flat = jnp.pad(verts, ((0, Vp - Vn), (0, 0))).reshape(3 * Vp)

    verts8 = _expand_verts(verts_flat, Vp)
    fv = _build_fv(faces_flat, verts8, Fp)
    res, vf, bq = _shade(pix, bary_lin, fv, N, HW)

    results = (res.reshape(B, H, 4, 4, 128)
               .transpose(0, 1, 2, 4, 3).reshape(B, H, W, 4))
    def unplanar(x):
        return (x.reshape(B, 3, H // 8, W // 128, 8, 128)
                .transpose(0, 2, 4, 3, 5, 1).reshape(B, H, W, 3))

    return (results, unplanar(vf), unplanar(bq))


# speedup vs baseline: 12.6680x; 1.0451x over previous
import functools

import jax
import jax.numpy as jnp
from jax import lax
from jax.experimental import pallas as pl
from jax.experimental.pallas import tpu as pltpu
from jax.experimental.pallas import tpu_sc as plsc

NW = 32
_PARAMS = pltpu.CompilerParams(
    use_tc_tiling_on_sc=False, needs_layout_passes=False)
_MESH = dict(core_axis_name="c", subcore_axis_name="s")


def _wid():
    return lax.axis_index("s") * 2 + lax.axis_index("c")


def _i16(v):
    return jnp.full((16,), v, jnp.int32)


def _expand_verts(verts_flat, Vp):
    mv = Vp // NW
    mesh = plsc.VectorSubcoreMesh(**_MESH)

    @functools.partial(
        pl.kernel,
        mesh=mesh,
        out_type=jax.ShapeDtypeStruct((Vp, 8), jnp.float32),
        compiler_params=_PARAMS,
        scratch_types=[
            pltpu.VMEM((3 * mv,), jnp.float32),
            pltpu.VMEM((mv, 8), jnp.float32),
        ],
    )
    def expand(vf_hbm, v8_hbm, vin_v, v8_v):
        base = _wid() * mv

        def group(g, _):
            lanes = g * 16 + lax.iota(jnp.int32, 16)
            for c in range(3):
                val = plsc.load_gather(vin_v, [3 * lanes + _i16(c)])
                plsc.store_scatter(v8_v, [lanes, _i16(c)], val)
            return 0

        pltpu.sync_copy(vf_hbm.at[pl.ds(3 * base, 3 * mv)], vin_v)
        lax.fori_loop(0, mv // 16, group, 0)
        pltpu.sync_copy(v8_v, v8_hbm.at[pl.ds(base, mv)])

    return expand(verts_flat)


def _build_fv(faces_flat, verts8, Fp):
    mf = Fp // NW
    rows3 = 3 * mf
    hf = mf // 2
    mesh = plsc.VectorSubcoreMesh(**_MESH)

    @functools.partial(
        pl.kernel,
        mesh=mesh,
        out_type=jax.ShapeDtypeStruct((Fp, 16), jnp.float32),
        compiler_params=_PARAMS,
        scratch_types=[
            pltpu.VMEM((rows3,), jnp.int32),
            pltpu.VMEM((rows3, 8), jnp.float32),
            pltpu.VMEM((hf, 16), jnp.float32),
            pltpu.SemaphoreType.DMA,
        ],
    )
    def build(ff_hbm, v8_hbm, fv_hbm, idx_v, rows_v, fv_v, sem):
        base = _wid() * mf
        pltpu.sync_copy(ff_hbm.at[pl.ds(3 * base, rows3)], idx_v)
        pltpu.async_copy(v8_hbm.at[idx_v], rows_v, sem).wait()
        for h in range(2):
            def group(g, _):
                lanes = g * 16 + lax.iota(jnp.int32, 16)
                rbase = 3 * (h * hf + lanes)
                for j in range(3):
                    for c in range(3):
                        val = plsc.load_gather(rows_v, [rbase + _i16(j), _i16(c)])
                        plsc.store_scatter(fv_v, [lanes, _i16(3 * j + c)], val)
                    ids = plsc.load_gather(idx_v, [rbase + _i16(j)])
                    plsc.store_scatter(fv_v, [lanes, _i16(9 + j)],
                                       plsc.bitcast(ids, jnp.float32))
                return 0

            lax.fori_loop(0, hf // 16, group, 0)
            pltpu.sync_copy(fv_v, fv_hbm.at[pl.ds(base + h * hf, hf)])

    return build(faces_flat, verts8)


def _shade(pix, bary_lin, fv, N, HW):
    m = 4096
    n_per = N // NW
    mesh = plsc.VectorSubcoreMesh(**_MESH)

    @functools.partial(
        pl.kernel,
        mesh=mesh,
        out_type=(
            jax.ShapeDtypeStruct((4 * N,), jnp.float32),
            jax.ShapeDtypeStruct((3 * N,), jnp.int32),
            jax.ShapeDtypeStruct((3 * N,), jnp.float32),
        ),
        compiler_params=_PARAMS,
        scratch_types=[
            pltpu.VMEM((m,), jnp.int32),
            pltpu.VMEM((m, 16), jnp.float32),
            pltpu.VMEM((3 * m,), jnp.float32),
            pltpu.VMEM((4 * m,), jnp.float32),
            pltpu.VMEM((3 * m,), jnp.int32),
            pltpu.VMEM((3 * m,), jnp.float32),
            pltpu.SemaphoreType.DMA,
            pltpu.SemaphoreType.DMA,
            pltpu.SemaphoreType.DMA,
        ],
    )
    def shade(pix_hbm, bary_hbm, fv_hbm, res_hbm, vfo_hbm, bq_hbm,
              pix_v, fv_v, bary_v, res_v, vf_v, bq_v, semA, semB, osem):
        wbase = _wid() * n_per
        hm = m // 2

        def out_copies(base):
            b_idx = base // HW
            inb = base % HW
            cps = [pltpu.make_async_copy(res_v, res_hbm.at[pl.ds(4 * base, 4 * m)],
                                         osem)]
            for j in range(3):
                dst = (3 * b_idx + j) * HW + inb
                cps.append(pltpu.make_async_copy(
                    vf_v.at[pl.ds(j * m, m)], vfo_hbm.at[pl.ds(dst, m)], osem))
                cps.append(pltpu.make_async_copy(
                    bq_v.at[pl.ds(j * m, m)], bq_hbm.at[pl.ds(dst, m)], osem))
            return cps

        def chunk(i, _):
            base = wbase + i * m
            pltpu.sync_copy(pix_hbm.at[pl.ds(base, m)], pix_v)
            pltpu.async_copy(fv_hbm.at[pix_v.at[pl.ds(0, hm)]],
                             fv_v.at[pl.ds(0, hm)], semA)
            pltpu.async_copy(fv_hbm.at[pix_v.at[pl.ds(hm, hm)]],
                             fv_v.at[pl.ds(hm, hm)], semB)
            pltpu.sync_copy(bary_hbm.at[pl.ds(3 * base, 3 * m)], bary_v)

            @pl.when(i > 0)
            def _drain_prev():
                for cp in out_copies(base - m):
                    cp.wait()

            def group(g, _):
                s = g * 16
                hh = s // 512
                sw = s % 512
                wt = sw // 128
                w7 = sw % 128
                rows = s + lax.iota(jnp.int32, 16)
                pv = pix_v[pl.ds(s, 16)]
                pl_off = wt * 1024 + hh * 128 + w7
                b = []
                for j in range(3):
                    bj = bary_v[pl.ds(hh * 1536 + j * 512 + sw, 16)]
                    b.append(bj)
                    bq_v[pl.ds(j * 4096 + pl_off, 16)] = bj
                for c in range(3):
                    acc = b[0] * plsc.load_gather(fv_v, [rows, _i16(c)])
                    for j in (1, 2):
                        acc = acc + b[j] * plsc.load_gather(
                            fv_v, [rows, _i16(3 * j + c)])
                    res_v[pl.ds(hh * 2048 + wt * 512 + c * 128 + w7, 16)] = acc
                alpha = jnp.where(pv != -1, 1.0, 0.0).astype(jnp.float32)
                res_v[pl.ds(hh * 2048 + wt * 512 + 3 * 128 + w7, 16)] = alpha
                for j in range(3):
                    ids = plsc.bitcast(
                        plsc.load_gather(fv_v, [rows, _i16(9 + j)]), jnp.int32)
                    vf_v[pl.ds(j * 4096 + pl_off, 16)] = ids
                return 0

            pltpu.make_async_copy(
                fv_hbm.at[pix_v.at[pl.ds(0, hm)]],
                fv_v.at[pl.ds(0, hm)], semA).wait()
            lax.fori_loop(0, hm // 16, group, 0)
            pltpu.make_async_copy(
                fv_hbm.at[pix_v.at[pl.ds(hm, hm)]],
                fv_v.at[pl.ds(hm, hm)], semB).wait()
            lax.fori_loop(hm // 16, m // 16, group, 0)
            for cp in out_copies(base):
                cp.start()
            return 0

        nchunks = n_per // m
        lax.fori_loop(0, nchunks, chunk, 0)
        for cp in out_copies(wbase + (nchunks - 1) * m):
            cp.wait()

    return shade(pix, bary_lin, fv)


def kernel(pix_to_face, bary_coords, faces, verts):
    B, H, W, _ = pix_to_face.shape
    N = B * H * W
    HW = H * W
    Fn = faces.shape[0]
    Vn = verts.shape[0]

    pix = pix_to_face.reshape(N)
    bary_lin = bary_coords.transpose(0, 1, 4, 3, 2).reshape(3 * N)

    Fp = -(-Fn // (NW * 32)) * (NW * 32)
    Vp = -(-Vn // (NW * 32)) * (NW * 32)
    faces_flat = jnp.pad(faces, ((0, Fp - Fn), (0, 0))).reshape(3 * Fp)
    verts_flat = jnp.pad(verts, ((0, Vp - Vn), (0, 0))).reshape(3 * Vp)

    verts8 = _expand_verts(verts_flat, Vp)
    fv = _build_fv(faces_flat, verts8, Fp)
    res, vf, bq = _shade(pix, bary_lin, fv, N, HW)

    results = (res.reshape(B, H, 4, 4, 128)
               .transpose(0, 1, 2, 4, 3).reshape(B, H, W, 4))
    def unplanar(x):
        return (x.reshape(B, 3, H // 8, W // 128, 8, 128)
                .transpose(0, 2, 4, 3, 5, 1).reshape(B, H, W, 3))

    return (results, unplanar(vf), unplanar(bq))


# speedup vs baseline: 15.3152x; 1.2090x over previous
import functools

import jax
import jax.numpy as jnp
from jax import lax
from jax.experimental import pallas as pl
from jax.experimental.pallas import tpu as pltpu
from jax.experimental.pallas import tpu_sc as plsc

NW = 32
_PARAMS = pltpu.CompilerParams(
    use_tc_tiling_on_sc=False, needs_layout_passes=False)
_MESH = dict(core_axis_name="c", subcore_axis_name="s")


def _wid():
    return lax.axis_index("s") * 2 + lax.axis_index("c")


def _i16(v):
    return jnp.full((16,), v, jnp.int32)


def _expand_verts(verts_flat, Vp):
    mv = Vp // NW
    mesh = plsc.VectorSubcoreMesh(**_MESH)

    @functools.partial(
        pl.kernel,
        mesh=mesh,
        out_type=jax.ShapeDtypeStruct((Vp, 8), jnp.float32),
        compiler_params=_PARAMS,
        scratch_types=[
            pltpu.VMEM((3 * mv,), jnp.float32),
            pltpu.VMEM((mv, 8), jnp.float32),
        ],
    )
    def expand(vf_hbm, v8_hbm, vin_v, v8_v):
        base = _wid() * mv

        def group(g):
            lanes = g * 16 + lax.iota(jnp.int32, 16)
            for c in range(3):
                val = plsc.load_gather(vin_v, [3 * lanes + _i16(c)])
                plsc.store_scatter(v8_v, [lanes, _i16(c)], val)

        pltpu.sync_copy(vf_hbm.at[pl.ds(3 * base, 3 * mv)], vin_v)
        plsc.parallel_loop(0, mv // 16, unroll=4)(group)
        pltpu.sync_copy(v8_v, v8_hbm.at[pl.ds(base, mv)])

    return expand(verts_flat)


def _build_fv(faces_flat, verts8, Fp):
    mf = Fp // NW
    rows3 = 3 * mf
    hf = mf // 2
    mesh = plsc.VectorSubcoreMesh(**_MESH)

    @functools.partial(
        pl.kernel,
        mesh=mesh,
        out_type=jax.ShapeDtypeStruct((Fp, 16), jnp.float32),
        compiler_params=_PARAMS,
        scratch_types=[
            pltpu.VMEM((rows3,), jnp.int32),
            pltpu.VMEM((rows3, 8), jnp.float32),
            pltpu.VMEM((hf, 16), jnp.float32),
            pltpu.SemaphoreType.DMA,
        ],
    )
    def build(ff_hbm, v8_hbm, fv_hbm, idx_v, rows_v, fv_v, sem):
        base = _wid() * mf
        pltpu.sync_copy(ff_hbm.at[pl.ds(3 * base, rows3)], idx_v)
        pltpu.async_copy(v8_hbm.at[idx_v], rows_v, sem).wait()
        for h in range(2):
            def group(g):
                lanes = g * 16 + lax.iota(jnp.int32, 16)
                rbase = 3 * (h * hf + lanes)
                for j in range(3):
                    for c in range(3):
                        val = plsc.load_gather(rows_v, [rbase + _i16(j), _i16(c)])
                        plsc.store_scatter(fv_v, [lanes, _i16(3 * j + c)], val)
                    ids = plsc.load_gather(idx_v, [rbase + _i16(j)])
                    plsc.store_scatter(fv_v, [lanes, _i16(9 + j)],
                                       plsc.bitcast(ids, jnp.float32))

            plsc.parallel_loop(0, hf // 16, unroll=2)(group)
            pltpu.sync_copy(fv_v, fv_hbm.at[pl.ds(base + h * hf, hf)])

    return build(faces_flat, verts8)


def _shade(pix, bary_lin, fv, N, HW):
    m = 4096
    n_per = N // NW
    mesh = plsc.VectorSubcoreMesh(**_MESH)

    @functools.partial(
        pl.kernel,
        mesh=mesh,
        out_type=(
            jax.ShapeDtypeStruct((4 * N,), jnp.float32),
            jax.ShapeDtypeStruct((3 * N,), jnp.int32),
            jax.ShapeDtypeStruct((3 * N,), jnp.float32),
        ),
        compiler_params=_PARAMS,
        scratch_types=[
            pltpu.VMEM((m,), jnp.int32),
            pltpu.VMEM((m, 16), jnp.float32),
            pltpu.VMEM((3 * m,), jnp.float32),
            pltpu.VMEM((4 * m,), jnp.float32),
            pltpu.VMEM((3 * m,), jnp.int32),
            pltpu.VMEM((3 * m,), jnp.float32),
            pltpu.SemaphoreType.DMA,
            pltpu.SemaphoreType.DMA,
            pltpu.SemaphoreType.DMA,
        ],
    )
    def shade(pix_hbm, bary_hbm, fv_hbm, res_hbm, vfo_hbm, bq_hbm,
              pix_v, fv_v, bary_v, res_v, vf_v, bq_v, semA, semB, osem):
        wbase = _wid() * n_per
        hm = m // 2

        def out_copies(base):
            b_idx = base // HW
            inb = base % HW
            cps = [pltpu.make_async_copy(res_v, res_hbm.at[pl.ds(4 * base, 4 * m)],
                                         osem)]
            for j in range(3):
                dst = (3 * b_idx + j) * HW + inb
                cps.append(pltpu.make_async_copy(
                    vf_v.at[pl.ds(j * m, m)], vfo_hbm.at[pl.ds(dst, m)], osem))
                cps.append(pltpu.make_async_copy(
                    bq_v.at[pl.ds(j * m, m)], bq_hbm.at[pl.ds(dst, m)], osem))
            return cps

        def chunk(i, _):
            base = wbase + i * m
            pltpu.sync_copy(pix_hbm.at[pl.ds(base, m)], pix_v)
            pltpu.async_copy(fv_hbm.at[pix_v.at[pl.ds(0, hm)]],
                             fv_v.at[pl.ds(0, hm)], semA)
            pltpu.async_copy(fv_hbm.at[pix_v.at[pl.ds(hm, hm)]],
                             fv_v.at[pl.ds(hm, hm)], semB)
            pltpu.sync_copy(bary_hbm.at[pl.ds(3 * base, 3 * m)], bary_v)

            @pl.when(i > 0)
            def _drain_prev():
                for cp in out_copies(base - m):
                    cp.wait()

            def group(g):
                s = g * 16
                hh = s // 512
                sw = s % 512
                wt = sw // 128
                w7 = sw % 128
                rows = s + lax.iota(jnp.int32, 16)
                pv = pix_v[pl.ds(s, 16)]
                pl_off = wt * 1024 + hh * 128 + w7
                b = []
                for j in range(3):
                    bj = bary_v[pl.ds(hh * 1536 + j * 512 + sw, 16)]
                    b.append(bj)
                    bq_v[pl.ds(j * 4096 + pl_off, 16)] = bj
                for c in range(3):
                    acc = b[0] * plsc.load_gather(fv_v, [rows, _i16(c)])
                    for j in (1, 2):
                        acc = acc + b[j] * plsc.load_gather(
                            fv_v, [rows, _i16(3 * j + c)])
                    res_v[pl.ds(hh * 2048 + wt * 512 + c * 128 + w7, 16)] = acc
                alpha = jnp.where(pv != -1, 1.0, 0.0).astype(jnp.float32)
                res_v[pl.ds(hh * 2048 + wt * 512 + 3 * 128 + w7, 16)] = alpha
                for j in range(3):
                    ids = plsc.bitcast(
                        plsc.load_gather(fv_v, [rows, _i16(9 + j)]), jnp.int32)
                    vf_v[pl.ds(j * 4096 + pl_off, 16)] = ids

            pltpu.make_async_copy(
                fv_hbm.at[pix_v.at[pl.ds(0, hm)]],
                fv_v.at[pl.ds(0, hm)], semA).wait()
            plsc.parallel_loop(0, hm // 16, unroll=4)(group)
            pltpu.make_async_copy(
                fv_hbm.at[pix_v.at[pl.ds(hm, hm)]],
                fv_v.at[pl.ds(hm, hm)], semB).wait()
            plsc.parallel_loop(hm // 16, m // 16, unroll=4)(group)
            for cp in out_copies(base):
                cp.start()
            return 0

        nchunks = n_per // m
        lax.fori_loop(0, nchunks, chunk, 0)
        for cp in out_copies(wbase + (nchunks - 1) * m):
            cp.wait()

    return shade(pix, bary_lin, fv)


def kernel(pix_to_face, bary_coords, faces, verts):
    B, H, W, _ = pix_to_face.shape
    N = B * H * W
    HW = H * W
    Fn = faces.shape[0]
    Vn = verts.shape[0]

    pix = pix_to_face.reshape(N)
    bary_lin = bary_coords.transpose(0, 1, 4, 3, 2).reshape(3 * N)

    Fp = -(-Fn // (NW * 32)) * (NW * 32)
    Vp = -(-Vn // (NW * 32)) * (NW * 32)
    faces_flat = jnp.pad(faces, ((0, Fp - Fn), (0, 0))).reshape(3 * Fp)
    verts_flat = jnp.pad(verts, ((0, Vp - Vn), (0, 0))).reshape(3 * Vp)

    verts8 = _expand_verts(verts_flat, Vp)
    fv = _build_fv(faces_flat, verts8, Fp)
    res, vf, bq = _shade(pix, bary_lin, fv, N, HW)

    results = (res.reshape(B, H, 4, 4, 128)
               .transpose(0, 1, 2, 4, 3).reshape(B, H, W, 4))
    def unplanar(x):
        return (x.reshape(B, 3, H // 8, W // 128, 8, 128)
                .transpose(0, 2, 4, 3, 5, 1).reshape(B, H, W, 3))

    return (results, unplanar(vf), unplanar(bq))
